# Initial kernel scaffold; baseline (speedup 1.0000x reference)
import jax
import jax.numpy as jnp
from jax.experimental import pallas as pl


def kernel(position_ids, d_0, d_1, d_2):
    raise NotImplementedError("write your pallas kernel here")


# speedup vs baseline: 1.4303x; 1.4303x over previous
import functools

import jax
import jax.numpy as jnp
from jax import lax
from jax.experimental import pallas as pl
from jax.experimental.pallas import tpu as pltpu
from jax.experimental.pallas import tpu_sc as plsc

B = 8192
D = 512
OUT_D = 3 * D
NW = 32
PW = B // NW
CHUNK = 64
LANES = 16


def _body(pos_hbm, d0_hbm, d1_hbm, d2_hbm, out_hbm, pos_v, idx_v, obuf, sem):
    cid = lax.axis_index("c")
    sid = lax.axis_index("s")
    wid = sid * 2 + cid
    base = wid * PW

    pltpu.sync_copy(pos_hbm.at[pl.ds(base, PW)], pos_v)

    for j in range(PW // LANES):
        sl = pl.ds(j * LANES, LANES)
        p = pos_v[sl]
        idx_v[0, sl] = lax.shift_right_logical(p, 10)
        idx_v[1, sl] = jnp.bitwise_and(lax.shift_right_logical(p, 5), 31)
        idx_v[2, sl] = jnp.bitwise_and(p, 31)

    tables = (d0_hbm, d1_hbm, d2_hbm)
    for c in range(PW // CHUNK):
        copies = []
        for s in range(3):
            copies.append(
                pltpu.async_copy(
                    tables[s].at[idx_v.at[s, pl.ds(c * CHUNK, CHUNK)]],
                    obuf.at[:, pl.ds(s * D, D)],
                    sem,
                )
            )
        for cp in copies:
            cp.wait()
        pltpu.sync_copy(obuf, out_hbm.at[pl.ds(base + c * CHUNK, CHUNK)])


@jax.jit
def _run(position_ids, d_0, d_1, d_2):
    mesh = plsc.VectorSubcoreMesh(core_axis_name="c", subcore_axis_name="s")
    kern = functools.partial(
        pl.kernel,
        out_type=jax.ShapeDtypeStruct((B, OUT_D), jnp.float32),
        mesh=mesh,
        scratch_types=[
            pltpu.VMEM((PW,), jnp.int32),
            pltpu.VMEM((3, PW), jnp.int32),
            pltpu.VMEM((CHUNK, OUT_D), jnp.float32),
            pltpu.SemaphoreType.DMA,
        ],
    )(_body)
    return kern(position_ids.astype(jnp.int32), d_0, d_1, d_2)


def kernel(position_ids, d_0, d_1, d_2):
    out = _run(position_ids, d_0, d_1, d_2)
    return out[None]
